# baseline (speedup 1.0000x reference)
import jax
import jax.numpy as jnp
from jax.experimental import pallas as pl
from jax.experimental.pallas import tpu as pltpu

E = 64
TOPK = 8
H = 1024
I = 768
IH = I // 4
HH = H // 4


def _moe_body(x_ref, gw_ref, w1a_ref, w1b_ref, w1c_ref, w1d_ref,
              w2a_ref, w2b_ref, w2c_ref, w2d_ref, out_ref, comb_ref):
    e = pl.program_id(0)
    T = x_ref.shape[0]
    xt = x_ref[...]

    @pl.when(e == 0)
    def _router():
        logits = jax.lax.dot_general(
            xt, gw_ref[...], (((1,), (1,)), ((), ())),
            preferred_element_type=jnp.float32)
        m = jnp.max(logits, axis=-1, keepdims=True)
        ex = jnp.exp(logits - m)
        probs = ex / jnp.sum(ex, axis=-1, keepdims=True)
        col = jax.lax.broadcasted_iota(jnp.int32, (T, E), 1)
        work = probs
        comb = jnp.zeros_like(probs)
        ssum = jnp.zeros((T, 1), jnp.float32)
        for _ in range(TOPK):
            mx = jnp.max(work, axis=-1, keepdims=True)
            sel = jnp.where(work == mx, col, E)
            j = jnp.min(sel, axis=-1, keepdims=True)
            chosen = col == j
            comb = comb + jnp.where(chosen, probs, 0.0)
            ssum = ssum + mx
            work = jnp.where(chosen, -jnp.inf, work)
        comb_ref[...] = comb / ssum

    ecol = jax.lax.broadcasted_iota(jnp.int32, (T, E), 1)
    c = jnp.sum(jnp.where(ecol == e, comb_ref[...], 0.0), axis=-1,
                keepdims=True)
    xb = xt.astype(jnp.bfloat16)

    def act(w1_ref):
        h = jax.lax.dot_general(
            xb, w1_ref[0].astype(jnp.bfloat16), (((1,), (1,)), ((), ())),
            preferred_element_type=jnp.float32)
        return ((h * jax.nn.sigmoid(h)) * c).astype(jnp.bfloat16)

    a = jnp.concatenate(
        [act(w1a_ref), act(w1b_ref), act(w1c_ref), act(w1d_ref)],
        axis=1)

    def down(w2_ref):
        return jax.lax.dot_general(
            a, w2_ref[0].astype(jnp.bfloat16), (((1,), (1,)), ((), ())),
            preferred_element_type=jnp.float32)

    y = jnp.concatenate(
        [down(w2a_ref), down(w2b_ref), down(w2c_ref), down(w2d_ref)],
        axis=1)

    @pl.when(e == 0)
    def _init():
        out_ref[...] = y

    @pl.when(e > 0)
    def _acc():
        out_ref[...] += y


def kernel(x, gate_w, w1, w2):
    orig_shape = x.shape
    xt = x.reshape(-1, x.shape[-1])
    T = xt.shape[0]
    out = pl.pallas_call(
        _moe_body,
        grid=(E,),
        in_specs=[
            pl.BlockSpec((T, H), lambda e: (0, 0)),
            pl.BlockSpec((E, H), lambda e: (0, 0)),
            pl.BlockSpec((1, IH, H), lambda e: (e, 0, 0)),
            pl.BlockSpec((1, IH, H), lambda e: (e, 1, 0)),
            pl.BlockSpec((1, IH, H), lambda e: (e, 2, 0)),
            pl.BlockSpec((1, IH, H), lambda e: (e, 3, 0)),
            pl.BlockSpec((1, HH, I), lambda e: (e, 0, 0)),
            pl.BlockSpec((1, HH, I), lambda e: (e, 1, 0)),
            pl.BlockSpec((1, HH, I), lambda e: (e, 2, 0)),
            pl.BlockSpec((1, HH, I), lambda e: (e, 3, 0)),
        ],
        out_specs=pl.BlockSpec((T, H), lambda e: (0, 0)),
        out_shape=jax.ShapeDtypeStruct((T, H), jnp.float32),
        scratch_shapes=[pltpu.VMEM((T, E), jnp.float32)],
        compiler_params=pltpu.CompilerParams(
            dimension_semantics=("arbitrary",)),
    )(xt, gate_w, w1, w1, w1, w1, w2, w2, w2, w2)
    return out.reshape(orig_shape)


# speedup vs baseline: 1.1272x; 1.1272x over previous
import jax
import jax.numpy as jnp
from jax.experimental import pallas as pl
from jax.experimental.pallas import tpu as pltpu

E = 64
TOPK = 8
H = 1024
I = 768
IH = I // 2
HH = H // 2
EPB = 2


def _moe_body(x_ref, gw_ref, w1a_ref, w1b_ref, w2a_ref, w2b_ref,
              out_ref, comb_ref):
    g = pl.program_id(0)
    T = x_ref.shape[0]
    xt = x_ref[...]

    @pl.when(g == 0)
    def _router():
        logits = jax.lax.dot_general(
            xt, gw_ref[...], (((1,), (1,)), ((), ())),
            preferred_element_type=jnp.float32)
        m = jnp.max(logits, axis=-1, keepdims=True)
        ex = jnp.exp(logits - m)
        probs = ex / jnp.sum(ex, axis=-1, keepdims=True)
        col = jax.lax.broadcasted_iota(jnp.int32, (T, E), 1)
        work = probs
        comb = jnp.zeros_like(probs)
        ssum = jnp.zeros((T, 1), jnp.float32)
        for _ in range(TOPK):
            mx = jnp.max(work, axis=-1, keepdims=True)
            sel = jnp.where(work == mx, col, E)
            j = jnp.min(sel, axis=-1, keepdims=True)
            chosen = col == j
            comb = comb + jnp.where(chosen, probs, 0.0)
            ssum = ssum + mx
            work = jnp.where(chosen, -jnp.inf, work)
        comb_ref[...] = comb / ssum

    ecol = jax.lax.broadcasted_iota(jnp.int32, (T, E), 1)
    xb = xt.astype(jnp.bfloat16)

    y = jnp.zeros((T, H), jnp.float32)
    for u in range(EPB):
        e = g * EPB + u
        c = jnp.sum(jnp.where(ecol == e, comb_ref[...], 0.0), axis=-1,
                    keepdims=True)

        def act(w1_ref, u=u, c=c):
            h = jax.lax.dot_general(
                xb, w1_ref[u].astype(jnp.bfloat16),
                (((1,), (1,)), ((), ())),
                preferred_element_type=jnp.float32)
            return ((h * jax.nn.sigmoid(h)) * c).astype(jnp.bfloat16)

        a = jnp.concatenate([act(w1a_ref), act(w1b_ref)], axis=1)

        def down(w2_ref, u=u, a=a):
            return jax.lax.dot_general(
                a, w2_ref[u].astype(jnp.bfloat16),
                (((1,), (1,)), ((), ())),
                preferred_element_type=jnp.float32)

        y = y + jnp.concatenate([down(w2a_ref), down(w2b_ref)], axis=1)

    @pl.when(g == 0)
    def _init():
        out_ref[...] = y

    @pl.when(g > 0)
    def _acc():
        out_ref[...] += y


def kernel(x, gate_w, w1, w2):
    orig_shape = x.shape
    xt = x.reshape(-1, x.shape[-1])
    T = xt.shape[0]
    out = pl.pallas_call(
        _moe_body,
        grid=(E // EPB,),
        in_specs=[
            pl.BlockSpec((T, H), lambda g: (0, 0)),
            pl.BlockSpec((E, H), lambda g: (0, 0)),
            pl.BlockSpec((EPB, IH, H), lambda g: (g, 0, 0)),
            pl.BlockSpec((EPB, IH, H), lambda g: (g, 1, 0)),
            pl.BlockSpec((EPB, HH, I), lambda g: (g, 0, 0)),
            pl.BlockSpec((EPB, HH, I), lambda g: (g, 1, 0)),
        ],
        out_specs=pl.BlockSpec((T, H), lambda g: (0, 0)),
        out_shape=jax.ShapeDtypeStruct((T, H), jnp.float32),
        scratch_shapes=[pltpu.VMEM((T, E), jnp.float32)],
        compiler_params=pltpu.CompilerParams(
            dimension_semantics=("arbitrary",)),
    )(xt, gate_w, w1, w1, w2, w2)
    return out.reshape(orig_shape)
